# Initial kernel scaffold; baseline (speedup 1.0000x reference)
import jax
import jax.numpy as jnp
from jax.experimental import pallas as pl


def kernel(feat, segment_ids, weight, bias, mean_scale):
    raise NotImplementedError("write your pallas kernel here")


# speedup vs baseline: 6.2781x; 6.2781x over previous
import functools

import jax
import jax.numpy as jnp
from jax import lax
from jax.experimental import pallas as pl
from jax.experimental.pallas import tpu as pltpu

N = 100000
D = 128
B = 512
K = 1000
NB = N // K


def _stats_body(seg_ref, feat_ref, sums_ref, sumsq_ref, cnt_ref):
    i = pl.program_id(0)
    seg = seg_ref[0, 0, :]
    feat = feat_ref[...]
    bi = lax.broadcasted_iota(jnp.int32, (B, K), 0)
    oh = (bi == seg[None, :]).astype(jnp.float32)
    rhs = jnp.concatenate([feat, feat * feat], axis=1)
    part = jnp.dot(oh, rhs, preferred_element_type=jnp.float32)
    part_cnt = jnp.broadcast_to(jnp.sum(oh, axis=1, keepdims=True), (B, 128))

    @pl.when(i == 0)
    def _():
        sums_ref[...] = part[:, :D]
        sumsq_ref[...] = part[:, D:]
        cnt_ref[...] = part_cnt

    @pl.when(i > 0)
    def _():
        sums_ref[...] += part[:, :D]
        sumsq_ref[...] += part[:, D:]
        cnt_ref[...] += part_cnt


def _apply_body(seg_ref, feat_ref, sums_ref, sumsq_ref, cnt_ref,
                w_ref, b_ref, ms_ref, out_ref, a_sc, c_sc):
    i = pl.program_id(0)

    @pl.when(i == 0)
    def _():
        cnt = jnp.maximum(cnt_ref[:, 0:1], 1.0)
        mean = sums_ref[...] / cnt
        m = mean * ms_ref[...]
        var = sumsq_ref[...] / cnt - m * (2.0 * mean - m)
        rstd = lax.rsqrt(var + 1e-6)
        a = w_ref[...] * rstd
        a_sc[...] = a
        c_sc[...] = b_ref[...] - m * a

    seg = seg_ref[0, 0, :]
    bi = lax.broadcasted_iota(jnp.int32, (K, B), 1)
    oh = (seg[:, None] == bi).astype(jnp.float32)
    a_pn = jnp.dot(oh, a_sc[...], preferred_element_type=jnp.float32)
    c_pn = jnp.dot(oh, c_sc[...], preferred_element_type=jnp.float32)
    out_ref[...] = feat_ref[...] * a_pn + c_pn


@jax.jit
def kernel(feat, segment_ids, weight, bias, mean_scale):
    seg3 = segment_ids.astype(jnp.int32).reshape(NB, 1, K)
    w2 = weight.reshape(1, D)
    b2 = bias.reshape(1, D)
    ms2 = mean_scale.reshape(1, D)

    seg_spec = pl.BlockSpec((1, 1, K), lambda i: (i, 0, 0))
    feat_spec = pl.BlockSpec((K, D), lambda i: (i, 0))
    full_spec = pl.BlockSpec((B, D), lambda i: (0, 0))
    cnt_spec = pl.BlockSpec((B, 128), lambda i: (0, 0))
    vec_spec = pl.BlockSpec((1, D), lambda i: (0, 0))

    sums, sumsq, cnt = pl.pallas_call(
        _stats_body,
        grid=(NB,),
        in_specs=[seg_spec, feat_spec],
        out_specs=[full_spec, full_spec, cnt_spec],
        out_shape=[
            jax.ShapeDtypeStruct((B, D), jnp.float32),
            jax.ShapeDtypeStruct((B, D), jnp.float32),
            jax.ShapeDtypeStruct((B, 128), jnp.float32),
        ],
    )(seg3, feat)

    out = pl.pallas_call(
        _apply_body,
        grid=(NB,),
        in_specs=[seg_spec, feat_spec, full_spec, full_spec, cnt_spec,
                  vec_spec, vec_spec, vec_spec],
        out_specs=pl.BlockSpec((K, D), lambda i: (i, 0)),
        out_shape=jax.ShapeDtypeStruct((N, D), jnp.float32),
        scratch_shapes=[pltpu.VMEM((B, D), jnp.float32),
                        pltpu.VMEM((B, D), jnp.float32)],
    )(seg3, feat, sums, sumsq, cnt, w2, b2, ms2)
    return out
